# baseline (speedup 1.0000x reference)
import jax
import jax.numpy as jnp
from jax.experimental import pallas as pl
from jax.experimental.pallas import tpu as pltpu

_P = 4194304
_CHUNK = 1 << 18
_NCHUNK = _P // _CHUNK
_TOTAL = 2 * _NCHUNK


def _body(a_smem, b_any, out_vmem, inb, sems):
    bi = pl.program_id(0)
    j = pl.program_id(1)
    i = bi * _NCHUNK + j

    def in_copy(b_idx, j_idx, slot):
        return pltpu.make_async_copy(
            b_any.at[b_idx, pl.ds(0, 1), pl.ds(j_idx * _CHUNK, _CHUNK)],
            inb.at[slot],
            sems.at[slot],
        )

    @pl.when(i == 0)
    def _():
        @pl.when(a_smem[0] != 0.0)
        def _():
            in_copy(0, 0, 0).start()

    i1 = i + 1
    b1 = jnp.minimum(i1 // _NCHUNK, 1)
    j1 = i1 % _NCHUNK

    @pl.when(jnp.logical_and(i1 < _TOTAL, a_smem[b1] != 0.0))
    def _():
        in_copy(b1, j1, i1 % 2).start()

    a = a_smem[bi]

    @pl.when(a != 0.0)
    def _():
        in_copy(bi, j, i % 2).wait()
        out_vmem[0] = a * inb[i % 2]

    @pl.when(a == 0.0)
    def _():
        out_vmem[0] = jnp.zeros((1, _CHUNK), jnp.float32)


def kernel(B, A):
    a2 = A.reshape(2)
    out = pl.pallas_call(
        _body,
        grid=(2, _NCHUNK),
        in_specs=[
            pl.BlockSpec(memory_space=pltpu.SMEM),
            pl.BlockSpec(memory_space=pl.ANY),
        ],
        out_specs=pl.BlockSpec((1, 1, _CHUNK), lambda b, j: (b, 0, j)),
        out_shape=jax.ShapeDtypeStruct((2, 1, _P), jnp.float32),
        scratch_shapes=[
            pltpu.VMEM((2, 1, _CHUNK), jnp.float32),
            pltpu.SemaphoreType.DMA((2,)),
        ],
    )(a2, B)
    return out


# speedup vs baseline: 1.4706x; 1.4706x over previous
import jax
import jax.numpy as jnp
from jax.experimental import pallas as pl
from jax.experimental.pallas import tpu as pltpu

_P = 4194304
_CHUNK = 1 << 20
_NCHUNK = _P // _CHUNK
_TOTAL = 2 * _NCHUNK


def _body(a_smem, b_any, out_vmem, inb, sems):
    bi = pl.program_id(0)
    j = pl.program_id(1)
    i = bi * _NCHUNK + j

    def in_copy(b_idx, j_idx, slot):
        return pltpu.make_async_copy(
            b_any.at[b_idx, pl.ds(0, 1), pl.ds(j_idx * _CHUNK, _CHUNK)],
            inb.at[slot],
            sems.at[slot],
        )

    @pl.when(i == 0)
    def _():
        @pl.when(a_smem[0] != 0.0)
        def _():
            in_copy(0, 0, 0).start()

    i1 = i + 1
    b1 = jnp.minimum(i1 // _NCHUNK, 1)
    j1 = i1 % _NCHUNK

    @pl.when(jnp.logical_and(i1 < _TOTAL, a_smem[b1] != 0.0))
    def _():
        in_copy(b1, j1, i1 % 2).start()

    a = a_smem[bi]

    @pl.when(a != 0.0)
    def _():
        in_copy(bi, j, i % 2).wait()
        out_vmem[0] = a * inb[i % 2]

    @pl.when(a == 0.0)
    def _():
        out_vmem[0] = jnp.zeros((1, _CHUNK), jnp.float32)


def kernel(B, A):
    a2 = A.reshape(2)
    out = pl.pallas_call(
        _body,
        grid=(2, _NCHUNK),
        in_specs=[
            pl.BlockSpec(memory_space=pltpu.SMEM),
            pl.BlockSpec(memory_space=pl.ANY),
        ],
        out_specs=pl.BlockSpec((1, 1, _CHUNK), lambda b, j: (b, 0, j)),
        out_shape=jax.ShapeDtypeStruct((2, 1, _P), jnp.float32),
        scratch_shapes=[
            pltpu.VMEM((2, 1, _CHUNK), jnp.float32),
            pltpu.SemaphoreType.DMA((2,)),
        ],
    )(a2, B)
    return out


# speedup vs baseline: 1.5209x; 1.0342x over previous
import jax
import jax.numpy as jnp
from jax.experimental import pallas as pl
from jax.experimental.pallas import tpu as pltpu

_P = 4194304
_CHUNK = 1 << 21
_NCHUNK = _P // _CHUNK
_TOTAL = 2 * _NCHUNK


def _body(a_smem, b_any, out_vmem, inb, sems):
    bi = pl.program_id(0)
    j = pl.program_id(1)
    i = bi * _NCHUNK + j

    def in_copy(b_idx, j_idx, slot):
        return pltpu.make_async_copy(
            b_any.at[b_idx, pl.ds(0, 1), pl.ds(j_idx * _CHUNK, _CHUNK)],
            inb.at[slot],
            sems.at[slot],
        )

    @pl.when(i == 0)
    def _():
        @pl.when(a_smem[0] != 0.0)
        def _():
            in_copy(0, 0, 0).start()

    i1 = i + 1
    b1 = jnp.minimum(i1 // _NCHUNK, 1)
    j1 = i1 % _NCHUNK

    @pl.when(jnp.logical_and(i1 < _TOTAL, a_smem[b1] != 0.0))
    def _():
        in_copy(b1, j1, i1 % 2).start()

    a = a_smem[bi]

    @pl.when(a != 0.0)
    def _():
        in_copy(bi, j, i % 2).wait()
        out_vmem[0] = a * inb[i % 2]

    @pl.when(a == 0.0)
    def _():
        out_vmem[0] = jnp.zeros((1, _CHUNK), jnp.float32)


def kernel(B, A):
    a2 = A.reshape(2)
    out = pl.pallas_call(
        _body,
        grid=(2, _NCHUNK),
        in_specs=[
            pl.BlockSpec(memory_space=pltpu.SMEM),
            pl.BlockSpec(memory_space=pl.ANY),
        ],
        out_specs=pl.BlockSpec((1, 1, _CHUNK), lambda b, j: (b, 0, j)),
        out_shape=jax.ShapeDtypeStruct((2, 1, _P), jnp.float32),
        scratch_shapes=[
            pltpu.VMEM((2, 1, _CHUNK), jnp.float32),
            pltpu.SemaphoreType.DMA((2,)),
        ],
    )(a2, B)
    return out
